# Initial kernel scaffold; baseline (speedup 1.0000x reference)
import jax
import jax.numpy as jnp
from jax.experimental import pallas as pl


def kernel(hidden_states, prefix_key_states, prefix_value_states, topk_k, Wq, Wk, Wv, Wo):
    raise NotImplementedError("write your pallas kernel here")


# speedup vs baseline: 48.0227x; 48.0227x over previous
import functools
import math

import jax
import jax.numpy as jnp
from jax import lax
from jax.experimental import pallas as pl
from jax.experimental.pallas import tpu as pltpu

_TOPK = 128
_BQ = 256

def _dot(a, b, dims):
    return lax.dot_general(a, b, dims, preferred_element_type=jnp.float32)


def _sortable(x_f32):
    b = lax.bitcast_convert_type(x_f32, jnp.int32)
    return b ^ jnp.where(b < 0, jnp.int32(2 ** 31 - 1), jnp.int32(0))


def _kth_threshold(skey, k):
    rows = skey.shape[0]
    minint = jnp.int32(-(2 ** 31))
    tu = jnp.zeros((rows, 1), jnp.int32)
    for i in range(32):
        bit = minint if i == 0 else jnp.int32(1 << (31 - i))
        cand = tu | bit
        cnt = jnp.sum((skey >= (cand ^ minint)).astype(jnp.int32), axis=1,
                      keepdims=True)
        tu = jnp.where(cnt >= k, cand, tu)
    return tu ^ minint


def _attn_body(hs_ref, pk_ref, pv_ref, wq_ref, wk_ref, wv_ref, o_ref,
               k_sc, v_sc, *, scale, bq, nq):
    qb = pl.program_id(1)

    @pl.when(qb == 0)
    def _project_kv():
        dn = (((1,), (1,)), ((), ()))
        k_sc[...] = _dot(hs_ref[...], wk_ref[...], dn)
        v_sc[...] = _dot(hs_ref[...], wv_ref[...], dn)

    dn = (((1,), (1,)), ((), ()))
    hs_q = hs_ref[pl.ds(qb * bq, bq), :]
    q = _dot(hs_q, wq_ref[...], dn)

    sp = _dot(q, pk_ref[...], dn) * scale
    skey = _sortable(sp)
    thr = _kth_threshold(skey, _TOPK)
    ep = jnp.where(skey >= thr, jnp.exp(sp), 0.0)
    dp = jnp.sum(ep, axis=1, keepdims=True)

    sd = _dot(q, k_sc[...], dn) * scale
    rows = qb * bq + lax.broadcasted_iota(jnp.int32, (bq, nq), 0)
    cols = lax.broadcasted_iota(jnp.int32, (bq, nq), 1)
    ed = jnp.where(cols <= rows, jnp.exp(sd), 0.0)
    dd = jnp.sum(ed, axis=1, keepdims=True)

    den = dp + dd
    op = _dot(ep / den, pv_ref[...], (((1,), (0,)), ((), ())))
    od = _dot(ed / den, v_sc[...], (((1,), (0,)), ((), ())))
    o_ref[...] = op + od


def _wo_body(a_ref, w_ref, o_ref):
    o_ref[...] = _dot(a_ref[...], w_ref[...], (((1,), (1,)), ((), ())))


def kernel(hidden_states, prefix_key_states, prefix_value_states, topk_k,
           Wq, Wk, Wv, Wo):
    b, nq, hid = hidden_states.shape
    _, h, npre, d = prefix_key_states.shape
    scale = 1.0 / math.sqrt(d)
    bq = _BQ
    nqb = nq // bq

    hs = hidden_states[0]
    pk = prefix_key_states[0]
    pv = prefix_value_states[0]
    wq3 = Wq.reshape(h, d, hid)
    wk3 = Wk.reshape(h, d, hid)
    wv3 = Wv.reshape(h, d, hid)

    attn = pl.pallas_call(
        functools.partial(_attn_body, scale=scale, bq=bq, nq=nq),
        grid=(h, nqb),
        in_specs=[
            pl.BlockSpec((nq, hid), lambda hh, qq: (0, 0)),
            pl.BlockSpec((None, npre, d), lambda hh, qq: (hh, 0, 0)),
            pl.BlockSpec((None, npre, d), lambda hh, qq: (hh, 0, 0)),
            pl.BlockSpec((None, d, hid), lambda hh, qq: (hh, 0, 0)),
            pl.BlockSpec((None, d, hid), lambda hh, qq: (hh, 0, 0)),
            pl.BlockSpec((None, d, hid), lambda hh, qq: (hh, 0, 0)),
        ],
        out_specs=pl.BlockSpec((bq, d), lambda hh, qq: (qq, hh)),
        out_shape=jax.ShapeDtypeStruct((nq, h * d), jnp.float32),
        scratch_shapes=[
            pltpu.VMEM((nq, d), jnp.float32),
            pltpu.VMEM((nq, d), jnp.float32),
        ],
    )(hs, pk, pv, wq3, wk3, wv3)

    attn2d = attn

    out = pl.pallas_call(
        _wo_body,
        grid=(nqb,),
        in_specs=[
            pl.BlockSpec((bq, h * d), lambda i: (i, 0)),
            pl.BlockSpec((hid, h * d), lambda i: (0, 0)),
        ],
        out_specs=pl.BlockSpec((bq, hid), lambda i: (i, 0)),
        out_shape=jax.ShapeDtypeStruct((nq, hid), jnp.float32),
    )(attn2d, Wo)

    return out[None]


# speedup vs baseline: 60.1134x; 1.2518x over previous
import functools
import math

import jax
import jax.numpy as jnp
from jax import lax
from jax.experimental import pallas as pl
from jax.experimental.pallas import tpu as pltpu

_TOPK = 128
_BQ = 256

def _dot(a, b, dims):
    return lax.dot_general(a, b, dims, preferred_element_type=jnp.float32)


def _sortable(x_f32):
    b = lax.bitcast_convert_type(x_f32, jnp.int32)
    return b ^ jnp.where(b < 0, jnp.int32(2 ** 31 - 1), jnp.int32(0))


_BITS = 24


def _kth_threshold(skey, k):
    rows = skey.shape[0]
    minint = jnp.int32(-(2 ** 31))
    kf = jnp.float32(k)
    tu = jnp.zeros((rows, 1), jnp.int32)
    for i in range(_BITS):
        bit = minint if i == 0 else jnp.int32(1 << (31 - i))
        cand = tu | bit
        cnt = jnp.sum(jnp.where(skey >= (cand ^ minint), 1.0, 0.0), axis=1,
                      keepdims=True)
        tu = jnp.where(cnt >= kf, cand, tu)
    return tu ^ minint


def _attn_body(hs_ref, pk_ref, pv_ref, wq_ref, wk_ref, wv_ref, o_ref,
               k_sc, v_sc, *, scale, bq, nq):
    qb = pl.program_id(1)

    @pl.when(qb == 0)
    def _project_kv():
        dn = (((1,), (1,)), ((), ()))
        k_sc[...] = _dot(hs_ref[...], wk_ref[...], dn)
        v_sc[...] = _dot(hs_ref[...], wv_ref[...], dn)

    dn = (((1,), (1,)), ((), ()))
    hs_q = hs_ref[pl.ds(qb * bq, bq), :]
    q = _dot(hs_q, wq_ref[...], dn)

    sp = _dot(q, pk_ref[...], dn) * scale
    skey = _sortable(sp)
    thr = _kth_threshold(skey, _TOPK)
    ep = jnp.where(skey >= thr, jnp.exp(sp), 0.0)
    dp = jnp.sum(ep, axis=1, keepdims=True)

    d_head = pv_ref.shape[-1]
    rows = lax.broadcasted_iota(jnp.int32, (bq, bq), 0)
    cols = lax.broadcasted_iota(jnp.int32, (bq, bq), 1)

    def _chunk(kb, carry):
        od_u, dd = carry
        ks = k_sc[pl.ds(kb * bq, bq), :]
        vs = v_sc[pl.ds(kb * bq, bq), :]
        s = _dot(q, ks, dn) * scale
        e = jnp.where((qb * bq + rows) >= (kb * bq + cols), jnp.exp(s), 0.0)
        od_u = od_u + _dot(e, vs, (((1,), (0,)), ((), ())))
        dd = dd + jnp.sum(e, axis=1, keepdims=True)
        return od_u, dd

    od_u, dd = lax.fori_loop(
        0, qb + 1, _chunk,
        (jnp.zeros((bq, d_head), jnp.float32), jnp.zeros((bq, 1), jnp.float32)))

    den = dp + dd
    op = _dot(ep / den, pv_ref[...], (((1,), (0,)), ((), ())))
    o_ref[...] = op + od_u / den


def _wo_body(a_ref, w_ref, o_ref):
    o_ref[...] = _dot(a_ref[...], w_ref[...], (((1,), (1,)), ((), ())))


def kernel(hidden_states, prefix_key_states, prefix_value_states, topk_k,
           Wq, Wk, Wv, Wo):
    b, nq, hid = hidden_states.shape
    _, h, npre, d = prefix_key_states.shape
    scale = 1.0 / math.sqrt(d)
    bq = _BQ
    nqb = nq // bq

    hs = hidden_states[0]
    pk = prefix_key_states[0]
    pv = prefix_value_states[0]
    wq3 = Wq.reshape(h, d, hid)
    wk3 = Wk.reshape(h, d, hid)
    wv3 = Wv.reshape(h, d, hid)

    attn = pl.pallas_call(
        functools.partial(_attn_body, scale=scale, bq=bq, nq=nq),
        grid=(h, nqb),
        in_specs=[
            pl.BlockSpec((nq, hid), lambda hh, qq: (0, 0)),
            pl.BlockSpec((None, npre, d), lambda hh, qq: (hh, 0, 0)),
            pl.BlockSpec((None, npre, d), lambda hh, qq: (hh, 0, 0)),
            pl.BlockSpec((None, d, hid), lambda hh, qq: (hh, 0, 0)),
            pl.BlockSpec((None, d, hid), lambda hh, qq: (hh, 0, 0)),
            pl.BlockSpec((None, d, hid), lambda hh, qq: (hh, 0, 0)),
        ],
        out_specs=pl.BlockSpec((bq, d), lambda hh, qq: (qq, hh)),
        out_shape=jax.ShapeDtypeStruct((nq, h * d), jnp.float32),
        scratch_shapes=[
            pltpu.VMEM((nq, d), jnp.float32),
            pltpu.VMEM((nq, d), jnp.float32),
        ],
    )(hs, pk, pv, wq3, wk3, wv3)

    attn2d = attn

    out = pl.pallas_call(
        _wo_body,
        grid=(nqb,),
        in_specs=[
            pl.BlockSpec((bq, h * d), lambda i: (i, 0)),
            pl.BlockSpec((hid, h * d), lambda i: (0, 0)),
        ],
        out_specs=pl.BlockSpec((bq, hid), lambda i: (i, 0)),
        out_shape=jax.ShapeDtypeStruct((nq, hid), jnp.float32),
    )(attn2d, Wo)

    return out[None]
